# Initial kernel scaffold; baseline (speedup 1.0000x reference)
import jax
import jax.numpy as jnp
from jax.experimental import pallas as pl


def kernel(center_words, context_words, negative_words, center_table, context_table):
    raise NotImplementedError("write your pallas kernel here")


# speedup vs baseline: 2.6591x; 2.6591x over previous
import functools

import jax
import jax.numpy as jnp
from jax import lax
from jax.experimental import pallas as pl
from jax.experimental.pallas import tpu as pltpu
from jax.experimental.pallas import tpu_sc as plsc

NC = 2
NS = 16
NW = NC * NS
LANES = 16
CB = 16


def _sc_body(B, N, D, cw_hbm, xw_hbm, nw_hbm, ctab_hbm, xtab_hbm, out_hbm,
             cw16, xw16, nwa, nwb, nwc, cen_v, ctx_v, neg_v, out_v,
             sem_i, sem_g):
  T = N + 1
  bpw = B // NW
  nchunk = bpw // CB
  wid = lax.axis_index("s") * NC + lax.axis_index("c")
  base = wid * bpw

  @pl.loop(0, nchunk)
  def _chunk(c):
    gbase = base + c * CB
    nb = gbase * N
    idx_copies = [
        pltpu.async_copy(cw_hbm.at[pl.ds(gbase, CB)], cw16, sem_i),
        pltpu.async_copy(xw_hbm.at[pl.ds(gbase, CB)], xw16, sem_i),
        pltpu.async_copy(nw_hbm.at[pl.ds(nb, 128)], nwa, sem_i),
        pltpu.async_copy(nw_hbm.at[pl.ds(nb + 128, 128)], nwb, sem_i),
        pltpu.async_copy(nw_hbm.at[pl.ds(nb + 256, 64)], nwc, sem_i),
    ]
    for cp in idx_copies:
      cp.wait()
    g_copies = [
        pltpu.async_copy(ctab_hbm.at[cw16], cen_v, sem_g),
        pltpu.async_copy(xtab_hbm.at[xw16], ctx_v, sem_g),
        pltpu.async_copy(xtab_hbm.at[nwa], neg_v.at[pl.ds(0, 128)], sem_g),
        pltpu.async_copy(xtab_hbm.at[nwb], neg_v.at[pl.ds(128, 128)], sem_g),
        pltpu.async_copy(xtab_hbm.at[nwc], neg_v.at[pl.ds(256, 64)], sem_g),
    ]
    for cp in g_copies:
      cp.wait()

    nreg = D // LANES

    @pl.loop(0, CB)
    def _b(b):
      c_regs = [cen_v[b, pl.ds(k * LANES, LANES)] for k in range(nreg)]
      acc = ctx_v[b, pl.ds(0, LANES)] * c_regs[0]
      for k in range(1, nreg):
        acc = acc + ctx_v[b, pl.ds(k * LANES, LANES)] * c_regs[k]
      out_v[b * T, :] = acc
      for t in range(N):
        r = b * N + t
        acc = neg_v[r, pl.ds(0, LANES)] * c_regs[0]
        for k in range(1, nreg):
          acc = acc + neg_v[r, pl.ds(k * LANES, LANES)] * c_regs[k]
        out_v[b * T + 1 + t, :] = acc

    pltpu.sync_copy(out_v, out_hbm.at[pl.ds(gbase * T, CB * T)])


def _sc_scores(cw, xw, nw_flat, ctab, xtab):
  B = cw.shape[0]
  N = nw_flat.shape[0] // B
  D = ctab.shape[1]
  T = N + 1
  mesh = plsc.VectorSubcoreMesh(core_axis_name="c", subcore_axis_name="s",
                                num_cores=NC, num_subcores=NS)
  body = functools.partial(_sc_body, B, N, D)
  f = pl.kernel(
      body,
      out_type=jax.ShapeDtypeStruct((B * T, LANES), jnp.float32),
      mesh=mesh,
      scratch_types=[
          pltpu.VMEM((CB,), jnp.int32),
          pltpu.VMEM((CB,), jnp.int32),
          pltpu.VMEM((128,), jnp.int32),
          pltpu.VMEM((128,), jnp.int32),
          pltpu.VMEM((64,), jnp.int32),
          pltpu.VMEM((CB, D), jnp.float32),
          pltpu.VMEM((CB, D), jnp.float32),
          pltpu.VMEM((CB * N, D), jnp.float32),
          pltpu.VMEM((CB * T, LANES), jnp.float32),
          pltpu.SemaphoreType.DMA,
          pltpu.SemaphoreType.DMA,
      ],
  )
  return f(cw, xw, nw_flat, ctab, xtab)


def _tc_loss_body(RB, T, inv_b, x_ref, o_ref):
  i = pl.program_id(0)
  x = x_ref[...]
  s = jnp.sum(x, axis=1, keepdims=True)
  rows = lax.broadcasted_iota(jnp.int32, s.shape, 0) + i * RB
  y = jnp.where(rows % T == 0, s, -s)
  sig = 1.0 / (1.0 + jnp.exp(-y))
  term = -jnp.log(sig + 1e-10)
  psum = jnp.sum(term) * inv_b

  @pl.when(i == 0)
  def _():
    o_ref[...] = jnp.zeros_like(o_ref)

  o_ref[...] += psum


def _tc_loss(part, B, T):
  R = part.shape[0]
  RB = 4096
  assert R % RB == 0
  grid = R // RB
  body = functools.partial(_tc_loss_body, RB, T, 1.0 / B)
  out = pl.pallas_call(
      body,
      grid=(grid,),
      in_specs=[pl.BlockSpec((RB, LANES), lambda i: (i, 0))],
      out_specs=pl.BlockSpec((1, 1), lambda i: (0, 0)),
      out_shape=jax.ShapeDtypeStruct((1, 1), jnp.float32),
  )(part)
  return out[0, 0]


def kernel(center_words, context_words, negative_words, center_table,
           context_table):
  B = center_words.shape[0]
  N = negative_words.shape[1]
  cw = center_words.astype(jnp.int32)
  xw = context_words.astype(jnp.int32)
  nw_flat = negative_words.astype(jnp.int32).reshape(-1)
  part = _sc_scores(cw, xw, nw_flat, center_table, context_table)
  return _tc_loss(part, B, N + 1)


# speedup vs baseline: 6.6535x; 2.5021x over previous
import functools

import jax
import jax.numpy as jnp
from jax import lax
from jax.experimental import pallas as pl
from jax.experimental.pallas import tpu as pltpu
from jax.experimental.pallas import tpu_sc as plsc

NC = 2
NS = 16
NW = NC * NS
LANES = 16
CB = 16


def _sc_body(B, N, D, cw_hbm, xw_hbm, nw_hbm, ctab_hbm, xtab_hbm, out_hbm,
             *scr):
  T = N + 1
  TP = 24
  orows = CB * TP * LANES // 128
  bpw = B // NW
  nchunk = bpw // CB
  nreg = D // LANES
  slots = [scr[0:9], scr[9:18]]
  sems = [scr[18:21], scr[21:24]]

  wid = lax.axis_index("s") * NC + lax.axis_index("c")
  base = wid * bpw

  def idx_pairs(slot, c):
    cw16, xw16, nwa, nwb, nwc = slots[slot][0:5]
    gbase = base + c * CB
    nb = gbase * N
    return [
        (cw_hbm.at[pl.ds(gbase, CB)], cw16),
        (xw_hbm.at[pl.ds(gbase, CB)], xw16),
        (nw_hbm.at[pl.ds(nb, 128)], nwa),
        (nw_hbm.at[pl.ds(nb + 128, 128)], nwb),
        (nw_hbm.at[pl.ds(nb + 256, 64)], nwc),
    ]

  def gather_pairs(slot):
    cw16, xw16, nwa, nwb, nwc, cen, ctx, neg = slots[slot][0:8]
    return [
        (ctab_hbm.at[cw16], cen),
        (xtab_hbm.at[xw16], ctx),
        (xtab_hbm.at[nwa], neg.at[pl.ds(0, 128)]),
        (xtab_hbm.at[nwb], neg.at[pl.ds(128, 128)]),
        (xtab_hbm.at[nwc], neg.at[pl.ds(256, 64)]),
    ]

  def out_pair(slot, c):
    gbase = base + c * CB
    off = pl.multiple_of(gbase * TP // 8, 8)
    return (slots[slot][8], out_hbm.at[pl.ds(off, orows)])

  def issue(pairs, sem):
    for s, d in pairs:
      pltpu.async_copy(s, d, sem)

  def drain(pairs, sem):
    for s, d in pairs:
      pltpu.make_async_copy(s, d, sem).wait()

  def compute(slot):
    cen, ctx, neg, out = slots[slot][5:9]

    @pl.loop(0, CB)
    def _b(b):
      c_regs = [cen[b, pl.ds(k * LANES, LANES)] for k in range(nreg)]

      def emit(src_ref, r, t):
        acc = src_ref[r, pl.ds(0, LANES)] * c_regs[0]
        for k in range(1, nreg):
          acc = acc + src_ref[r, pl.ds(k * LANES, LANES)] * c_regs[k]
        pair = b * TP + t
        out[pair // 8, pl.ds((pair % 8) * LANES, LANES)] = acc

      emit(ctx, b, 0)
      for t in range(N):
        emit(neg, b * N + t, 1 + t)

  def step(slot, c):
    other = 1 - slot

    @pl.when(c + 1 < nchunk)
    def _():
      drain(idx_pairs(other, c + 1), sems[other][0])
      issue(gather_pairs(other), sems[other][1])

    drain(gather_pairs(slot), sems[slot][1])

    @pl.when(c + 2 < nchunk)
    def _():
      issue(idx_pairs(slot, c + 2), sems[slot][0])

    @pl.when(c >= 2)
    def _():
      drain([out_pair(slot, c - 2)], sems[slot][2])

    compute(slot)
    issue([out_pair(slot, c)], sems[slot][2])

  zeros = jnp.zeros((LANES,), jnp.float32)
  for slot in range(2):
    out = slots[slot][8]
    for b in range(CB):
      for t in range(T, TP):
        pair = b * TP + t
        out[pair // 8, pl.ds((pair % 8) * LANES, LANES)] = zeros

  issue(idx_pairs(0, 0), sems[0][0])
  drain(idx_pairs(0, 0), sems[0][0])
  issue(gather_pairs(0), sems[0][1])
  issue(idx_pairs(1, 1), sems[1][0])

  @pl.loop(0, nchunk, step=2)
  def _pair(c0):
    step(0, c0)
    step(1, c0 + 1)

  drain([out_pair(0, nchunk - 2)], sems[0][2])
  drain([out_pair(1, nchunk - 1)], sems[1][2])


def _sc_scores(cw, xw, nw_flat, ctab, xtab):
  B = cw.shape[0]
  N = nw_flat.shape[0] // B
  D = ctab.shape[1]
  T = N + 1
  TP = 24
  orows = CB * TP * LANES // 128
  mesh = plsc.VectorSubcoreMesh(core_axis_name="c", subcore_axis_name="s",
                                num_cores=NC, num_subcores=NS)
  slot_scr = [
      pltpu.VMEM((CB,), jnp.int32),
      pltpu.VMEM((CB,), jnp.int32),
      pltpu.VMEM((128,), jnp.int32),
      pltpu.VMEM((128,), jnp.int32),
      pltpu.VMEM((64,), jnp.int32),
      pltpu.VMEM((CB, D), jnp.float32),
      pltpu.VMEM((CB, D), jnp.float32),
      pltpu.VMEM((CB * N, D), jnp.float32),
      pltpu.VMEM((orows, 128), jnp.float32),
  ]
  body = functools.partial(_sc_body, B, N, D)
  f = pl.kernel(
      body,
      out_type=jax.ShapeDtypeStruct((B * TP * LANES // 128, 128), jnp.float32),
      mesh=mesh,
      scratch_types=(slot_scr + slot_scr
                     + [pltpu.SemaphoreType.DMA] * 6),
  )
  return f(cw, xw, nw_flat, ctab, xtab)


def _tc_loss_body(RB, T, scale, x_ref, o_ref):
  i = pl.program_id(0)
  x = x_ref[...]
  gi = lax.broadcasted_iota(jnp.int32, (128, 128), 0) // LANES
  gj = lax.broadcasted_iota(jnp.int32, (128, 128), 1) // LANES
  gmask = (gi == gj).astype(jnp.float32)
  s = jnp.dot(x, gmask, preferred_element_type=jnp.float32)
  rows = lax.broadcasted_iota(jnp.int32, x.shape, 0) + i * RB
  lane_g = lax.broadcasted_iota(jnp.int32, x.shape, 1) // LANES
  q = (rows % 3) * 8 + lane_g
  y = jnp.where(q == 0, s, -s)
  sig = 1.0 / (1.0 + jnp.exp(-y))
  term = -jnp.log(sig + 1e-10)
  term = jnp.where(q < T, term, 0.0)
  psum = jnp.sum(term) * scale

  @pl.when(i == 0)
  def _():
    o_ref[...] = jnp.zeros_like(o_ref)

  o_ref[...] += psum


def _tc_loss(part, B, T):
  R8 = part.shape[0]
  RB = 2048
  assert R8 % RB == 0
  grid = R8 // RB
  body = functools.partial(_tc_loss_body, RB, T, 1.0 / (B * LANES))
  out = pl.pallas_call(
      body,
      grid=(grid,),
      in_specs=[pl.BlockSpec((RB, 128), lambda i: (i, 0))],
      out_specs=pl.BlockSpec((1, 1), lambda i: (0, 0)),
      out_shape=jax.ShapeDtypeStruct((1, 1), jnp.float32),
  )(part)
  return out[0, 0]


def kernel(center_words, context_words, negative_words, center_table,
           context_table):
  B = center_words.shape[0]
  N = negative_words.shape[1]
  cw = center_words.astype(jnp.int32)
  xw = context_words.astype(jnp.int32)
  nw_flat = negative_words.astype(jnp.int32).reshape(-1)
  part = _sc_scores(cw, xw, nw_flat, center_table, context_table)
  return _tc_loss(part, B, N + 1)
